# Initial kernel scaffold; baseline (speedup 1.0000x reference)
import jax
import jax.numpy as jnp
from jax.experimental import pallas as pl


def kernel(x, edge_index, W1, b1, W2, b2):
    raise NotImplementedError("write your pallas kernel here")


# speedup vs baseline: 30.8992x; 30.8992x over previous
import functools

import jax
import jax.numpy as jnp
from jax import lax
from jax.experimental import pallas as pl
from jax.experimental.pallas import tpu as pltpu
from jax.experimental.pallas import tpu_sc as plsc

N = 10000
E = 320000
D = 128
H = 16
NT = 10112
NC, NS = 2, 16
NW = NC * NS
CHUNK = 128
CPT = 79
EPT = CPT * CHUNK
EPAD = NW * EPT
RPT = NT // NS

_mesh = plsc.VectorSubcoreMesh(
    core_axis_name="c", subcore_axis_name="s", num_cores=NC, num_subcores=NS
)

_sc_params = pltpu.CompilerParams(use_tc_tiling_on_sc=False)

_f32 = jnp.float32


@functools.partial(
    pl.kernel,
    out_type=jax.ShapeDtypeStruct((NC, NT, H), _f32),
    mesh=_mesh,
    compiler_params=_sc_params,
    scratch_types=[
        pltpu.VMEM_SHARED((NT, H), _f32),
        pltpu.VMEM((CPT, CHUNK), jnp.int32),
        pltpu.VMEM((CHUNK, H), _f32),
    ],
)
def _sc_degree(dst_hbm, ones_hbm, zeros_hbm, out_hbm, acc_sh, dst_v, ones_v):
    c = lax.axis_index("c")
    s = lax.axis_index("s")
    wid = s * NC + c
    pltpu.sync_copy(dst_hbm.at[wid], dst_v)
    pltpu.sync_copy(ones_hbm, ones_v)
    pltpu.sync_copy(
        zeros_hbm.at[pl.ds(s * RPT, RPT)], acc_sh.at[pl.ds(s * RPT, RPT)]
    )
    plsc.subcore_barrier()

    @pl.loop(0, CPT)
    def _(j):
        pltpu.sync_copy(ones_v, acc_sh.at[dst_v.at[j]], add=True)

    plsc.subcore_barrier()
    pltpu.sync_copy(
        acc_sh.at[pl.ds(s * RPT, RPT)], out_hbm.at[c, pl.ds(s * RPT, RPT)]
    )


@functools.partial(
    pl.kernel,
    out_type=jax.ShapeDtypeStruct((NC, NT, H), _f32),
    mesh=_mesh,
    compiler_params=_sc_params,
    scratch_types=[
        pltpu.VMEM_SHARED((NT, H), _f32),
        pltpu.VMEM((CPT, CHUNK), jnp.int32),
        pltpu.VMEM((CPT, CHUNK), jnp.int32),
        pltpu.VMEM((CHUNK, H), _f32),
    ],
)
def _sc_gather_scatter(
    table_hbm, src_hbm, dst_hbm, zeros_hbm, out_hbm, acc_sh, src_v, dst_v, buf_v
):
    c = lax.axis_index("c")
    s = lax.axis_index("s")
    wid = s * NC + c
    pltpu.sync_copy(src_hbm.at[wid], src_v)
    pltpu.sync_copy(dst_hbm.at[wid], dst_v)
    pltpu.sync_copy(
        zeros_hbm.at[pl.ds(s * RPT, RPT)], acc_sh.at[pl.ds(s * RPT, RPT)]
    )
    plsc.subcore_barrier()

    @pl.loop(0, CPT)
    def _(j):
        pltpu.sync_copy(table_hbm.at[src_v.at[j]], buf_v)
        pltpu.sync_copy(buf_v, acc_sh.at[dst_v.at[j]], add=True)

    plsc.subcore_barrier()
    pltpu.sync_copy(
        acc_sh.at[pl.ds(s * RPT, RPT)], out_hbm.at[c, pl.ds(s * RPT, RPT)]
    )


def _tc_matmul_body(x_ref, w_ref, b_ref, h_ref):
    h_ref[...] = (
        jnp.dot(x_ref[...], w_ref[...], preferred_element_type=_f32) + b_ref[...]
    )


_tc_matmul = pl.pallas_call(
    _tc_matmul_body, out_shape=jax.ShapeDtypeStruct((NT, H), _f32)
)


def _tc_scale_body(degp_ref, h_ref, isd_ref, hn_ref):
    deg = degp_ref[0, :, 0:1] + degp_ref[1, :, 0:1] + 1.0
    isd = lax.rsqrt(deg)
    isd_ref[...] = isd
    hn_ref[...] = h_ref[...] * isd


_tc_scale = pl.pallas_call(
    _tc_scale_body,
    out_shape=(
        jax.ShapeDtypeStruct((NT, 1), _f32),
        jax.ShapeDtypeStruct((NT, H), _f32),
    ),
)


def _tc_layer2_body(aggp_ref, hn_ref, isd_ref, w2t_ref, b2_ref, g16_ref):
    isd = isd_ref[...]
    tot = (aggp_ref[0] + aggp_ref[1] + hn_ref[...]) * isd
    a1 = jnp.maximum(tot, 0.0)
    h2 = jnp.sum(a1 * w2t_ref[...], axis=1, keepdims=True) + b2_ref[...]
    g16_ref[...] = jnp.broadcast_to(h2 * isd, (NT, H))


_tc_layer2 = pl.pallas_call(
    _tc_layer2_body, out_shape=jax.ShapeDtypeStruct((NT, H), _f32)
)


def _tc_out_body(aggp2_ref, g16_ref, isd_ref, out_ref):
    tot = aggp2_ref[0, :, 0:1] + aggp2_ref[1, :, 0:1] + g16_ref[:, 0:1]
    out_ref[...] = jax.nn.sigmoid(tot * isd_ref[...])


_tc_out = pl.pallas_call(
    _tc_out_body, out_shape=jax.ShapeDtypeStruct((NT, 1), _f32)
)


@jax.jit
def kernel(x, edge_index, W1, b1, W2, b2):
    src = edge_index[0]
    dst = edge_index[1]
    srcp = jnp.concatenate(
        [src, jnp.zeros((EPAD - E,), jnp.int32)]
    ).reshape(NW, CPT, CHUNK)
    dstp = jnp.concatenate(
        [dst, jnp.full((EPAD - E,), N, jnp.int32)]
    ).reshape(NW, CPT, CHUNK)
    xp = jnp.pad(x, ((0, NT - N), (0, 0)))
    zeros16 = jnp.zeros((NT, H), _f32)
    ones16 = jnp.ones((CHUNK, H), _f32)

    degp = _sc_degree(dstp, ones16, zeros16)
    h = _tc_matmul(xp, W1, b1.reshape(1, H))
    isd, hn = _tc_scale(degp, h)
    aggp = _sc_gather_scatter(hn, srcp, dstp, zeros16)
    g16 = _tc_layer2(aggp, hn, isd, W2.reshape(1, H), b2.reshape(1, 1))
    aggp2 = _sc_gather_scatter(g16, srcp, dstp, zeros16)
    out = _tc_out(aggp2, g16, isd)
    return out[:N]


# speedup vs baseline: 48.9636x; 1.5846x over previous
import functools

import jax
import jax.numpy as jnp
from jax import lax
from jax.experimental import pallas as pl
from jax.experimental.pallas import tpu as pltpu
from jax.experimental.pallas import tpu_sc as plsc

N = 10000
E = 320000
D = 128
H = 16
NT = 10112
NC, NS = 2, 16
NW = NC * NS
CHUNK = 128
CPT = 79
EPT = CPT * CHUNK
EPAD = NW * EPT
RPT = NT // NS

_mesh = plsc.VectorSubcoreMesh(
    core_axis_name="c", subcore_axis_name="s", num_cores=NC, num_subcores=NS
)

_sc_params = pltpu.CompilerParams(use_tc_tiling_on_sc=False)

_f32 = jnp.float32


@functools.partial(
    pl.kernel,
    out_type=jax.ShapeDtypeStruct((NC, NT, H), _f32),
    mesh=_mesh,
    compiler_params=_sc_params,
    scratch_types=[
        pltpu.VMEM_SHARED((NT, H), _f32),
        pltpu.VMEM((CPT, CHUNK), jnp.int32),
        pltpu.VMEM((CHUNK, H), _f32),
    ],
)
def _sc_degree(dst_hbm, ones_hbm, zeros_hbm, out_hbm, acc_sh, dst_v, ones_v):
    c = lax.axis_index("c")
    s = lax.axis_index("s")
    wid = s * NC + c
    pltpu.sync_copy(dst_hbm.at[wid], dst_v)
    pltpu.sync_copy(ones_hbm, ones_v)
    pltpu.sync_copy(
        zeros_hbm.at[pl.ds(s * RPT, RPT)], acc_sh.at[pl.ds(s * RPT, RPT)]
    )
    plsc.subcore_barrier()

    @pl.loop(0, CPT)
    def _(j):
        pltpu.sync_copy(ones_v, acc_sh.at[dst_v.at[j]], add=True)

    plsc.subcore_barrier()
    pltpu.sync_copy(
        acc_sh.at[pl.ds(s * RPT, RPT)], out_hbm.at[c, pl.ds(s * RPT, RPT)]
    )


@functools.partial(
    pl.kernel,
    out_type=jax.ShapeDtypeStruct((NC, NT, H), _f32),
    mesh=_mesh,
    compiler_params=_sc_params,
    scratch_types=[
        pltpu.VMEM_SHARED((NT, H), _f32),
        pltpu.VMEM_SHARED((NT, H), _f32),
        pltpu.VMEM((CPT, CHUNK), jnp.int32),
        pltpu.VMEM((CPT, CHUNK), jnp.int32),
        pltpu.VMEM((CHUNK, H), _f32),
    ],
)
def _sc_gather_scatter(
    table_hbm, src_hbm, dst_hbm, zeros_hbm, out_hbm,
    acc_sh, tab_sh, src_v, dst_v, buf_v,
):
    c = lax.axis_index("c")
    s = lax.axis_index("s")
    wid = s * NC + c
    pltpu.sync_copy(src_hbm.at[wid], src_v)
    pltpu.sync_copy(dst_hbm.at[wid], dst_v)
    pltpu.sync_copy(
        zeros_hbm.at[pl.ds(s * RPT, RPT)], acc_sh.at[pl.ds(s * RPT, RPT)]
    )
    pltpu.sync_copy(
        table_hbm.at[pl.ds(s * RPT, RPT)], tab_sh.at[pl.ds(s * RPT, RPT)]
    )
    plsc.subcore_barrier()

    @pl.loop(0, CPT)
    def _(j):
        pltpu.sync_copy(tab_sh.at[src_v.at[j]], buf_v)
        pltpu.sync_copy(buf_v, acc_sh.at[dst_v.at[j]], add=True)

    plsc.subcore_barrier()
    pltpu.sync_copy(
        acc_sh.at[pl.ds(s * RPT, RPT)], out_hbm.at[c, pl.ds(s * RPT, RPT)]
    )


def _tc_matmul_body(x_ref, w_ref, b_ref, h_ref):
    h_ref[...] = (
        jnp.dot(x_ref[...], w_ref[...], preferred_element_type=_f32) + b_ref[...]
    )


_tc_matmul = pl.pallas_call(
    _tc_matmul_body, out_shape=jax.ShapeDtypeStruct((NT, H), _f32)
)


def _tc_scale_body(degp_ref, h_ref, isd_ref, hn_ref):
    deg = degp_ref[0, :, 0:1] + degp_ref[1, :, 0:1] + 1.0
    isd = lax.rsqrt(deg)
    isd_ref[...] = isd
    hn_ref[...] = h_ref[...] * isd


_tc_scale = pl.pallas_call(
    _tc_scale_body,
    out_shape=(
        jax.ShapeDtypeStruct((NT, 1), _f32),
        jax.ShapeDtypeStruct((NT, H), _f32),
    ),
)


def _tc_layer2_body(aggp_ref, hn_ref, isd_ref, w2t_ref, b2_ref, g16_ref):
    isd = isd_ref[...]
    tot = (aggp_ref[0] + aggp_ref[1] + hn_ref[...]) * isd
    a1 = jnp.maximum(tot, 0.0)
    h2 = jnp.sum(a1 * w2t_ref[...], axis=1, keepdims=True) + b2_ref[...]
    g16_ref[...] = jnp.broadcast_to(h2 * isd, (NT, H))


_tc_layer2 = pl.pallas_call(
    _tc_layer2_body, out_shape=jax.ShapeDtypeStruct((NT, H), _f32)
)


def _tc_out_body(aggp2_ref, g16_ref, isd_ref, out_ref):
    tot = aggp2_ref[0, :, 0:1] + aggp2_ref[1, :, 0:1] + g16_ref[:, 0:1]
    out_ref[...] = jax.nn.sigmoid(tot * isd_ref[...])


_tc_out = pl.pallas_call(
    _tc_out_body, out_shape=jax.ShapeDtypeStruct((NT, 1), _f32)
)


@jax.jit
def kernel(x, edge_index, W1, b1, W2, b2):
    src = edge_index[0]
    dst = edge_index[1]
    pad = jnp.arange(EPAD - E, dtype=jnp.int32)
    srcp = jnp.concatenate([src, pad % N]).reshape(NW, CPT, CHUNK)
    dstp = jnp.concatenate([dst, N + pad % (NT - N)]).reshape(NW, CPT, CHUNK)
    xp = jnp.pad(x, ((0, NT - N), (0, 0)))
    zeros16 = jnp.zeros((NT, H), _f32)
    ones16 = jnp.ones((CHUNK, H), _f32)

    degp = _sc_degree(dstp, ones16, zeros16)
    h = _tc_matmul(xp, W1, b1.reshape(1, H))
    isd, hn = _tc_scale(degp, h)
    aggp = _sc_gather_scatter(hn, srcp, dstp, zeros16)
    g16 = _tc_layer2(aggp, hn, isd, W2.reshape(1, H), b2.reshape(1, 1))
    aggp2 = _sc_gather_scatter(g16, srcp, dstp, zeros16)
    out = _tc_out(aggp2, g16, isd)
    return out[:N]


# speedup vs baseline: 53.2185x; 1.0869x over previous
import functools

import jax
import jax.numpy as jnp
from jax import lax
from jax.experimental import pallas as pl
from jax.experimental.pallas import tpu as pltpu
from jax.experimental.pallas import tpu_sc as plsc

N = 10000
E = 320000
D = 128
H = 16
HD = 8
NT = 10112
NC, NS = 2, 16
NW = NC * NS
CHUNK = 128
ECH = E // CHUNK
CPT = 79
CPT_LAST = ECH - 31 * CPT
RPT = NT // NS

_mesh = plsc.VectorSubcoreMesh(
    core_axis_name="c", subcore_axis_name="s", num_cores=NC, num_subcores=NS
)

_sc_params = pltpu.CompilerParams(use_tc_tiling_on_sc=False)

_f32 = jnp.float32


def _stage_indices(e_hbm, row, wid, idx_v):
    base = wid * CPT

    @pl.when(wid < NW - 1)
    def _():
        pltpu.sync_copy(e_hbm.at[row, pl.ds(base, CPT)], idx_v)

    @pl.when(wid == NW - 1)
    def _():
        pltpu.sync_copy(
            e_hbm.at[row, pl.ds(base, CPT_LAST)], idx_v.at[pl.ds(0, CPT_LAST)]
        )


@functools.partial(
    pl.kernel,
    out_type=jax.ShapeDtypeStruct((NC, NT, HD), _f32),
    mesh=_mesh,
    compiler_params=_sc_params,
    scratch_types=[
        pltpu.VMEM_SHARED((NT, HD), _f32),
        pltpu.VMEM((CPT, CHUNK), jnp.int32),
        pltpu.VMEM((CHUNK, HD), _f32),
    ],
)
def _sc_degree(e_hbm, ones_hbm, zeros_hbm, out_hbm, acc_sh, dst_v, ones_v):
    c = lax.axis_index("c")
    s = lax.axis_index("s")
    wid = s * NC + c
    nch = jnp.where(wid == NW - 1, CPT_LAST, CPT)
    _stage_indices(e_hbm, 1, wid, dst_v)
    pltpu.sync_copy(ones_hbm, ones_v)
    pltpu.sync_copy(
        zeros_hbm.at[pl.ds(s * RPT, RPT)], acc_sh.at[pl.ds(s * RPT, RPT)]
    )
    plsc.subcore_barrier()

    @pl.loop(0, CPT)
    def _(j):
        @pl.when(j < nch)
        def _():
            pltpu.sync_copy(ones_v, acc_sh.at[dst_v.at[j]], add=True)

    plsc.subcore_barrier()
    pltpu.sync_copy(
        acc_sh.at[pl.ds(s * RPT, RPT)], out_hbm.at[c, pl.ds(s * RPT, RPT)]
    )


@functools.partial(
    pl.kernel,
    out_type=jax.ShapeDtypeStruct((NC, NT, H), _f32),
    mesh=_mesh,
    compiler_params=_sc_params,
    scratch_types=[
        pltpu.VMEM_SHARED((NT, H), _f32),
        pltpu.VMEM_SHARED((NT, H), _f32),
        pltpu.VMEM((CPT, CHUNK), jnp.int32),
        pltpu.VMEM((CPT, CHUNK), jnp.int32),
        pltpu.VMEM((CHUNK, H), _f32),
    ],
)
def _sc_gather_scatter(
    table_hbm, e_hbm, zeros_hbm, out_hbm, acc_sh, tab_sh, src_v, dst_v, buf_v
):
    c = lax.axis_index("c")
    s = lax.axis_index("s")
    wid = s * NC + c
    nch = jnp.where(wid == NW - 1, CPT_LAST, CPT)
    _stage_indices(e_hbm, 0, wid, src_v)
    _stage_indices(e_hbm, 1, wid, dst_v)
    pltpu.sync_copy(
        zeros_hbm.at[pl.ds(s * RPT, RPT)], acc_sh.at[pl.ds(s * RPT, RPT)]
    )
    pltpu.sync_copy(
        table_hbm.at[pl.ds(s * RPT, RPT)], tab_sh.at[pl.ds(s * RPT, RPT)]
    )
    plsc.subcore_barrier()

    @pl.loop(0, CPT)
    def _(j):
        @pl.when(j < nch)
        def _():
            pltpu.sync_copy(tab_sh.at[src_v.at[j]], buf_v)
            pltpu.sync_copy(buf_v, acc_sh.at[dst_v.at[j]], add=True)

    plsc.subcore_barrier()
    pltpu.sync_copy(
        acc_sh.at[pl.ds(s * RPT, RPT)], out_hbm.at[c, pl.ds(s * RPT, RPT)]
    )


def _tc_mm_scale_body(x_ref, w_ref, b_ref, degp_ref, isd_ref, hn_ref):
    deg = degp_ref[0, :, 0:1] + degp_ref[1, :, 0:1] + 1.0
    isd = lax.rsqrt(deg)
    isd_ref[...] = isd
    h = jnp.dot(x_ref[...], w_ref[...], preferred_element_type=_f32) + b_ref[...]
    hn_ref[pl.ds(0, N), :] = h * isd[0:N, :]
    hn_ref[pl.ds(N, NT - N), :] = jnp.zeros((NT - N, H), _f32)


_tc_mm_scale = pl.pallas_call(
    _tc_mm_scale_body,
    out_shape=(
        jax.ShapeDtypeStruct((NT, 1), _f32),
        jax.ShapeDtypeStruct((NT, H), _f32),
    ),
)


def _tc_layer2_body(aggp_ref, hn_ref, isd_ref, w2t_ref, b2_ref, g16_ref):
    isd = isd_ref[...]
    tot = (aggp_ref[0] + aggp_ref[1] + hn_ref[...]) * isd
    a1 = jnp.maximum(tot, 0.0)
    h2 = jnp.sum(a1 * w2t_ref[...], axis=1, keepdims=True) + b2_ref[...]
    g16_ref[...] = jnp.broadcast_to(h2 * isd, (NT, H))


_tc_layer2 = pl.pallas_call(
    _tc_layer2_body, out_shape=jax.ShapeDtypeStruct((NT, H), _f32)
)


def _tc_out_body(aggp2_ref, g16_ref, isd_ref, out_ref):
    tot = aggp2_ref[0, :, 0:1] + aggp2_ref[1, :, 0:1] + g16_ref[:, 0:1]
    out_ref[...] = jax.nn.sigmoid(tot * isd_ref[...])


_tc_out = pl.pallas_call(
    _tc_out_body, out_shape=jax.ShapeDtypeStruct((NT, 1), _f32)
)


@jax.jit
def kernel(x, edge_index, W1, b1, W2, b2):
    e3 = edge_index.reshape(2, ECH, CHUNK)
    zeros16 = jnp.zeros((NT, H), _f32)
    zeros8 = jnp.zeros((NT, HD), _f32)
    ones8 = jnp.ones((CHUNK, HD), _f32)

    degp = _sc_degree(e3, ones8, zeros8)
    isd, hn = _tc_mm_scale(x, W1, b1.reshape(1, H), degp)
    aggp = _sc_gather_scatter(hn, e3, zeros16)
    g16 = _tc_layer2(aggp, hn, isd, W2.reshape(1, H), b2.reshape(1, 1))
    aggp2 = _sc_gather_scatter(g16, e3, zeros16)
    out = _tc_out(aggp2, g16, isd)
    return out[:N]


# speedup vs baseline: 55.1567x; 1.0364x over previous
import functools

import jax
import jax.numpy as jnp
from jax import lax
from jax.experimental import pallas as pl
from jax.experimental.pallas import tpu as pltpu
from jax.experimental.pallas import tpu_sc as plsc

N = 10000
E = 320000
D = 128
H = 16
HD = 8
NT = 10112
NC, NS = 2, 16
NW = NC * NS
CHUNK = 128
ECH = E // CHUNK
CPT = 79
CPT_LAST = ECH - 31 * CPT
RPT = NT // NS

_mesh = plsc.VectorSubcoreMesh(
    core_axis_name="c", subcore_axis_name="s", num_cores=NC, num_subcores=NS
)

_sc_params = pltpu.CompilerParams(use_tc_tiling_on_sc=False)

_f32 = jnp.float32


def _stage_indices(e_hbm, row, wid, idx_v):
    base = wid * CPT

    @pl.when(wid < NW - 1)
    def _():
        pltpu.sync_copy(e_hbm.at[row, pl.ds(base, CPT)], idx_v)

    @pl.when(wid == NW - 1)
    def _():
        pltpu.sync_copy(
            e_hbm.at[row, pl.ds(base, CPT_LAST)], idx_v.at[pl.ds(0, CPT_LAST)]
        )


@functools.partial(
    pl.kernel,
    out_type=jax.ShapeDtypeStruct((NC, NT, HD), _f32),
    mesh=_mesh,
    compiler_params=_sc_params,
    scratch_types=[
        pltpu.VMEM_SHARED((NT, HD), _f32),
        pltpu.VMEM((CPT, CHUNK), jnp.int32),
        pltpu.VMEM((CHUNK, HD), _f32),
    ],
)
def _sc_degree(e_hbm, ones_hbm, zeros_hbm, out_hbm, acc_sh, dst_v, ones_v):
    c = lax.axis_index("c")
    s = lax.axis_index("s")
    wid = s * NC + c
    nch = jnp.where(wid == NW - 1, CPT_LAST, CPT)
    _stage_indices(e_hbm, 1, wid, dst_v)
    pltpu.sync_copy(ones_hbm, ones_v)
    pltpu.sync_copy(
        zeros_hbm.at[pl.ds(s * RPT, RPT)], acc_sh.at[pl.ds(s * RPT, RPT)]
    )
    plsc.subcore_barrier()

    @pl.loop(0, CPT)
    def _(j):
        @pl.when(j < nch)
        def _():
            pltpu.sync_copy(ones_v, acc_sh.at[dst_v.at[j]], add=True)

    plsc.subcore_barrier()
    pltpu.sync_copy(
        acc_sh.at[pl.ds(s * RPT, RPT)], out_hbm.at[c, pl.ds(s * RPT, RPT)]
    )


def _make_gs(w):
    @functools.partial(
        pl.kernel,
        out_type=jax.ShapeDtypeStruct((NC, NT, w), _f32),
        mesh=_mesh,
        compiler_params=_sc_params,
        scratch_types=[
            pltpu.VMEM_SHARED((NT, w), _f32),
            pltpu.VMEM_SHARED((NT, w), _f32),
            pltpu.VMEM((CPT, CHUNK), jnp.int32),
            pltpu.VMEM((CPT, CHUNK), jnp.int32),
            pltpu.VMEM((CHUNK, w), _f32),
        ],
    )
    def gs(table_hbm, e_hbm, zeros_hbm, out_hbm, acc_sh, tab_sh, src_v, dst_v, buf_v):
        c = lax.axis_index("c")
        s = lax.axis_index("s")
        wid = s * NC + c
        nch = jnp.where(wid == NW - 1, CPT_LAST, CPT)
        _stage_indices(e_hbm, 0, wid, src_v)
        _stage_indices(e_hbm, 1, wid, dst_v)
        pltpu.sync_copy(
            zeros_hbm.at[pl.ds(s * RPT, RPT)], acc_sh.at[pl.ds(s * RPT, RPT)]
        )
        pltpu.sync_copy(
            table_hbm.at[pl.ds(s * RPT, RPT)], tab_sh.at[pl.ds(s * RPT, RPT)]
        )
        plsc.subcore_barrier()

        @pl.loop(0, CPT)
        def _(j):
            @pl.when(j < nch)
            def _():
                pltpu.sync_copy(tab_sh.at[src_v.at[j]], buf_v)
                pltpu.sync_copy(buf_v, acc_sh.at[dst_v.at[j]], add=True)

        plsc.subcore_barrier()
        pltpu.sync_copy(
            acc_sh.at[pl.ds(s * RPT, RPT)], out_hbm.at[c, pl.ds(s * RPT, RPT)]
        )

    return gs


_sc_gs16 = _make_gs(H)
_sc_gs8 = _make_gs(HD)


def _tc_mm_scale_body(x_ref, w_ref, b_ref, degp_ref, isd_ref, hn_ref):
    deg = degp_ref[0, :, 0:1] + degp_ref[1, :, 0:1] + 1.0
    isd = lax.rsqrt(deg)
    isd_ref[...] = isd
    h = jnp.dot(x_ref[...], w_ref[...], preferred_element_type=_f32) + b_ref[...]
    hn_ref[pl.ds(0, N), :] = h * isd[0:N, :]
    hn_ref[pl.ds(N, NT - N), :] = jnp.zeros((NT - N, H), _f32)


_tc_mm_scale = pl.pallas_call(
    _tc_mm_scale_body,
    out_shape=(
        jax.ShapeDtypeStruct((NT, 1), _f32),
        jax.ShapeDtypeStruct((NT, H), _f32),
    ),
)


def _tc_layer2_body(aggp_ref, hn_ref, isd_ref, w2t_ref, b2_ref, g8_ref):
    isd = isd_ref[...]
    tot = (aggp_ref[0] + aggp_ref[1] + hn_ref[...]) * isd
    a1 = jnp.maximum(tot, 0.0)
    h2 = jnp.sum(a1 * w2t_ref[...], axis=1, keepdims=True) + b2_ref[...]
    g8_ref[...] = jnp.broadcast_to(h2 * isd, (NT, HD))


_tc_layer2 = pl.pallas_call(
    _tc_layer2_body, out_shape=jax.ShapeDtypeStruct((NT, HD), _f32)
)


def _tc_out_body(aggp2_ref, g8_ref, isd_ref, out_ref):
    tot = aggp2_ref[0, :, 0:1] + aggp2_ref[1, :, 0:1] + g8_ref[:, 0:1]
    out_ref[...] = jax.nn.sigmoid(tot * isd_ref[...])


_tc_out = pl.pallas_call(
    _tc_out_body, out_shape=jax.ShapeDtypeStruct((NT, 1), _f32)
)


@jax.jit
def kernel(x, edge_index, W1, b1, W2, b2):
    e3 = edge_index.reshape(2, ECH, CHUNK)
    zeros16 = jnp.zeros((NT, H), _f32)
    zeros8 = jnp.zeros((NT, HD), _f32)
    ones8 = jnp.ones((CHUNK, HD), _f32)

    degp = _sc_degree(e3, ones8, zeros8)
    isd, hn = _tc_mm_scale(x, W1, b1.reshape(1, H), degp)
    aggp = _sc_gs16(hn, e3, zeros16)
    g8 = _tc_layer2(aggp, hn, isd, W2.reshape(1, H), b2.reshape(1, 1))
    aggp2 = _sc_gs8(g8, e3, zeros8)
    out = _tc_out(aggp2, g8, isd)
    return out[:N]


# speedup vs baseline: 55.2938x; 1.0025x over previous
import functools

import jax
import jax.numpy as jnp
from jax import lax
from jax.experimental import pallas as pl
from jax.experimental.pallas import tpu as pltpu
from jax.experimental.pallas import tpu_sc as plsc

N = 10000
E = 320000
D = 128
H = 16
NT = 10112
NC, NS = 2, 16
NW = NC * NS
CHUNK = 128
ECH = E // CHUNK
CPT = 79
CPT_LAST = ECH - 31 * CPT
RPT = NT // NS
RPW = NT // NW

_mesh = plsc.VectorSubcoreMesh(
    core_axis_name="c", subcore_axis_name="s", num_cores=NC, num_subcores=NS
)

_sc_params = pltpu.CompilerParams(
    use_tc_tiling_on_sc=False, needs_layout_passes=False
)

_f32 = jnp.float32


def _rsqrt(x):
    xi = lax.bitcast_convert_type(x, jnp.int32)
    y = lax.bitcast_convert_type(jnp.int32(0x5F3759DF) - (xi >> 1), _f32)
    xh = 0.5 * x
    y = y * (1.5 - xh * y * y)
    y = y * (1.5 - xh * y * y)
    y = y * (1.5 - xh * y * y)
    return y


def _stage_indices(e_hbm, row, wid, idx_v):
    base = wid * CPT

    @pl.when(wid < NW - 1)
    def _():
        pltpu.sync_copy(e_hbm.at[row, pl.ds(base, CPT)], idx_v)

    @pl.when(wid == NW - 1)
    def _():
        pltpu.sync_copy(
            e_hbm.at[row, pl.ds(base, CPT_LAST)], idx_v.at[pl.ds(0, CPT_LAST)]
        )


def _gs_loop(nch, src_v, dst_v, tab_sh, acc_sh, buf_v):
    @pl.loop(0, CPT)
    def _(j):
        @pl.when(j < nch)
        def _():
            pltpu.sync_copy(tab_sh.at[src_v.at[j]], buf_v)
            pltpu.sync_copy(buf_v, acc_sh.at[dst_v.at[j]], add=True)


@functools.partial(
    pl.kernel,
    out_type=jax.ShapeDtypeStruct((NC, NT, H), _f32),
    mesh=_mesh,
    compiler_params=_sc_params,
    scratch_types=[
        pltpu.VMEM_SHARED((NT, H), _f32),
        pltpu.VMEM((CPT, CHUNK), jnp.int32),
        pltpu.VMEM((CHUNK, H), _f32),
    ],
)
def _sc_degree(e_hbm, ones_hbm, zeros_hbm, out_hbm, acc_sh, dst_v, ones_v):
    c = lax.axis_index("c")
    s = lax.axis_index("s")
    wid = s * NC + c
    nch = jnp.where(wid == NW - 1, CPT_LAST, CPT)
    _stage_indices(e_hbm, 1, wid, dst_v)
    pltpu.sync_copy(ones_hbm, ones_v)
    pltpu.sync_copy(
        zeros_hbm.at[pl.ds(s * RPT, RPT)], acc_sh.at[pl.ds(s * RPT, RPT)]
    )
    plsc.subcore_barrier()

    @pl.loop(0, CPT)
    def _(j):
        @pl.when(j < nch)
        def _():
            pltpu.sync_copy(ones_v, acc_sh.at[dst_v.at[j]], add=True)

    plsc.subcore_barrier()
    pltpu.sync_copy(
        acc_sh.at[pl.ds(s * RPT, RPT)], out_hbm.at[c, pl.ds(s * RPT, RPT)]
    )


@functools.partial(
    pl.kernel,
    out_type=(
        jax.ShapeDtypeStruct((NC, NT, H), _f32),
        jax.ShapeDtypeStruct((NT, H), _f32),
        jax.ShapeDtypeStruct((NT, H), _f32),
    ),
    mesh=_mesh,
    compiler_params=_sc_params,
    scratch_types=[
        pltpu.VMEM_SHARED((NT, H), _f32),
        pltpu.VMEM_SHARED((NT, H), _f32),
        pltpu.VMEM((CPT, CHUNK), jnp.int32),
        pltpu.VMEM((CPT, CHUNK), jnp.int32),
        pltpu.VMEM((CHUNK, H), _f32),
        pltpu.VMEM((RPT, H), _f32),
        pltpu.VMEM((RPT, H), _f32),
        pltpu.VMEM((RPT, H), _f32),
    ],
)
def _sc_gs1(
    h_hbm, degp_hbm, e_hbm, zeros_hbm,
    agg_hbm, hn_hbm, isd_hbm,
    acc_sh, tab_sh, src_v, dst_v, buf_v, hb_v, d0_v, d1_v,
):
    c = lax.axis_index("c")
    s = lax.axis_index("s")
    wid = s * NC + c
    nch = jnp.where(wid == NW - 1, CPT_LAST, CPT)
    rows = pl.ds(s * RPT, RPT)
    _stage_indices(e_hbm, 0, wid, src_v)
    _stage_indices(e_hbm, 1, wid, dst_v)
    pltpu.sync_copy(zeros_hbm.at[rows], acc_sh.at[rows])
    pltpu.sync_copy(h_hbm.at[rows], hb_v)
    pltpu.sync_copy(degp_hbm.at[0, rows], d0_v)
    pltpu.sync_copy(degp_hbm.at[1, rows], d1_v)

    @pl.loop(0, RPT)
    def _(r):
        isd = _rsqrt(d0_v[r] + d1_v[r] + 1.0)
        d0_v[r] = isd
        hb_v[r] = hb_v[r] * isd

    pltpu.sync_copy(hb_v, tab_sh.at[rows])

    @pl.when(c == 0)
    def _():
        pltpu.sync_copy(hb_v, hn_hbm.at[rows])
        pltpu.sync_copy(d0_v, isd_hbm.at[rows])

    plsc.subcore_barrier()
    _gs_loop(nch, src_v, dst_v, tab_sh, acc_sh, buf_v)
    plsc.subcore_barrier()
    pltpu.sync_copy(acc_sh.at[rows], agg_hbm.at[c, rows])


@functools.partial(
    pl.kernel,
    out_type=(
        jax.ShapeDtypeStruct((NC, NT, H), _f32),
        jax.ShapeDtypeStruct((NT, H), _f32),
    ),
    mesh=_mesh,
    compiler_params=_sc_params,
    scratch_types=[
        pltpu.VMEM_SHARED((NT, H), _f32),
        pltpu.VMEM_SHARED((NT, H), _f32),
        pltpu.VMEM((CPT, CHUNK), jnp.int32),
        pltpu.VMEM((CPT, CHUNK), jnp.int32),
        pltpu.VMEM((CHUNK, H), _f32),
        pltpu.VMEM((RPT, H), _f32),
        pltpu.VMEM((RPT, H), _f32),
        pltpu.VMEM((RPT, H), _f32),
        pltpu.VMEM((RPT, H), _f32),
        pltpu.VMEM((2, H), _f32),
    ],
)
def _sc_gs2(
    aggp_hbm, hn_hbm, isd_hbm, wb_hbm, e_hbm, zeros_hbm,
    agg_hbm, g_hbm,
    acc_sh, tab_sh, src_v, dst_v, buf_v, a0_v, a1_v, hn_v, isd_v, wb_v,
):
    c = lax.axis_index("c")
    s = lax.axis_index("s")
    wid = s * NC + c
    nch = jnp.where(wid == NW - 1, CPT_LAST, CPT)
    rows = pl.ds(s * RPT, RPT)
    _stage_indices(e_hbm, 0, wid, src_v)
    _stage_indices(e_hbm, 1, wid, dst_v)
    pltpu.sync_copy(zeros_hbm.at[rows], acc_sh.at[rows])
    pltpu.sync_copy(aggp_hbm.at[0, rows], a0_v)
    pltpu.sync_copy(aggp_hbm.at[1, rows], a1_v)
    pltpu.sync_copy(hn_hbm.at[rows], hn_v)
    pltpu.sync_copy(isd_hbm.at[rows], isd_v)
    pltpu.sync_copy(wb_hbm, wb_v)

    @pl.loop(0, RPT)
    def _(r):
        isd = isd_v[r]
        a1 = jnp.maximum((a0_v[r] + a1_v[r] + hn_v[r]) * isd, 0.0)
        a0_v[r] = (a1 * wb_v[0] + wb_v[1]) * isd

    pltpu.sync_copy(a0_v, tab_sh.at[rows])

    @pl.when(c == 0)
    def _():
        pltpu.sync_copy(a0_v, g_hbm.at[rows])

    plsc.subcore_barrier()
    _gs_loop(nch, src_v, dst_v, tab_sh, acc_sh, buf_v)
    plsc.subcore_barrier()
    pltpu.sync_copy(acc_sh.at[rows], agg_hbm.at[c, rows])


@functools.partial(
    pl.kernel,
    out_type=jax.ShapeDtypeStruct((NT, H), _f32),
    mesh=_mesh,
    compiler_params=_sc_params,
    scratch_types=[
        pltpu.VMEM((RPW, H), _f32),
        pltpu.VMEM((RPW, H), _f32),
        pltpu.VMEM((RPW, H), _f32),
        pltpu.VMEM((RPW, H), _f32),
        pltpu.VMEM((H,), _f32),
    ],
)
def _sc_final(aggp_hbm, g_hbm, isd_hbm, out_hbm, a0_v, a1_v, g_v, isd_v, scr_v):
    c = lax.axis_index("c")
    s = lax.axis_index("s")
    wid = s * NC + c
    rows = pl.ds(wid * RPW, RPW)
    pltpu.sync_copy(aggp_hbm.at[0, rows], a0_v)
    pltpu.sync_copy(aggp_hbm.at[1, rows], a1_v)
    pltpu.sync_copy(g_hbm.at[rows], g_v)
    pltpu.sync_copy(isd_hbm.at[rows], isd_v)
    lane = jnp.arange(H, dtype=jnp.int32)

    @pl.loop(0, RPW)
    def _(r):
        t = a0_v[r] + a1_v[r] + g_v[r]
        for k in (8, 4, 2, 1):
            scr_v[...] = t
            t = t + plsc.load_gather(scr_v, [lane ^ k])
        t = t * isd_v[r]
        a0_v[r] = 1.0 / (1.0 + jnp.exp(-t))

    pltpu.sync_copy(a0_v, out_hbm.at[rows])


def _tc_mm_body(x_ref, w_ref, b_ref, h_ref):
    h = jnp.dot(x_ref[...], w_ref[...], preferred_element_type=_f32) + b_ref[...]
    h_ref[pl.ds(0, N), :] = h
    h_ref[pl.ds(N, NT - N), :] = jnp.zeros((NT - N, H), _f32)


_tc_mm = pl.pallas_call(_tc_mm_body, out_shape=jax.ShapeDtypeStruct((NT, H), _f32))


@jax.jit
def kernel(x, edge_index, W1, b1, W2, b2):
    e3 = edge_index.reshape(2, ECH, CHUNK)
    zeros16 = jnp.zeros((NT, H), _f32)
    ones16 = jnp.ones((CHUNK, H), _f32)
    wb = jnp.stack([W2.reshape(H), jnp.broadcast_to(b2.reshape(1) / H, (H,))])

    h = _tc_mm(x, W1, b1.reshape(1, H))
    degp = _sc_degree(e3, ones16, zeros16)
    aggp, hn, isd = _sc_gs1(h, degp, e3, zeros16)
    aggp2, g = _sc_gs2(aggp, hn, isd, wb, e3, zeros16)
    out = _sc_final(aggp2, g, isd)
    return out[:N, :1]
